# baseline (speedup 1.0000x reference)
import functools

import jax
import jax.numpy as jnp
from jax import lax
from jax.experimental import pallas as pl
from jax.experimental.pallas import tpu as pltpu
from jax.experimental.pallas import tpu_sc as plsc

NC = 2
NS = 16
LANES = 16


@functools.lru_cache(maxsize=None)
def _make_kernel(B, L, V, D):
    NW = NC * NS
    RPW = B // NW
    G = LANES
    NG = RPW // G
    HEAD = LANES - 1
    MAIN = L - HEAD
    assert B % (NW * G) == 0 and NG % 2 == 0 and L > HEAD
    DSTRIDE = L + 1
    assert DSTRIDE % 2 == 1 and V <= LANES

    mesh = plsc.VectorSubcoreMesh(core_axis_name="c", subcore_axis_name="s")

    @functools.partial(
        pl.kernel,
        out_type=jax.ShapeDtypeStruct((B * D,), jnp.float32),
        mesh=mesh,
        compiler_params=pltpu.CompilerParams(needs_layout_passes=False),
        scratch_types=[
            pltpu.VMEM((G * L,), jnp.int32),
            pltpu.VMEM((G * L,), jnp.int32),
            pltpu.VMEM((V * D * LANES,), jnp.float32),
            pltpu.VMEM((V * LANES,), jnp.float32),
            pltpu.VMEM((RPW * D,), jnp.float32),
            pltpu.SemaphoreType.DMA,
            pltpu.SemaphoreType.DMA,
        ],
    )
    def sc_kernel(ids_hbm, tab_hbm, out_hbm,
                  buf_a, buf_b, tab_v, counts, out_v, sem_a, sem_b):
        wid = lax.axis_index("s") * NC + lax.axis_index("c")
        base = wid * (RPW * L)

        pltpu.sync_copy(tab_hbm, tab_v)

        iota = lax.iota(jnp.int32, LANES)
        ones = jnp.full((LANES,), 1.0, jnp.float32)
        zeros = jnp.zeros((LANES,), jnp.float32)
        inv_l = jnp.full((LANES,), 1.0 / L, jnp.float32)
        diag = iota * DSTRIDE

        def dma(gi, buf, sem):
            return pltpu.make_async_copy(
                ids_hbm.at[pl.ds(base + gi * (G * L), G * L)], buf, sem)

        dma(0, buf_a, sem_a).start()
        dma(1, buf_b, sem_b).start()

        def process(buf, g):
            for k in range(V * LANES // LANES):
                counts[pl.ds(k * LANES, LANES)] = zeros

            for j in range(1, LANES):
                wrap = jnp.where(iota < j, L - j, -j).astype(jnp.int32)
                ids = plsc.load_gather(buf, [diag + wrap])
                plsc.addupdate_scatter(
                    counts, [lax.shift_left(ids, 4) + iota], ones)

            def hbody(_, idx):
                ids = plsc.load_gather(buf, [idx])
                plsc.addupdate_scatter(
                    counts, [lax.shift_left(ids, 4) + iota], ones)
                return idx + 1

            idx_f = lax.fori_loop(0, MAIN, hbody, diag, unroll=8)

            accs = [zeros] * D
            for v in range(V):
                row = counts[pl.ds(v * LANES, LANES)]
                for d in range(D):
                    accs[d] = accs[d] + row * tab_v[
                        pl.ds((v * D + d) * LANES, LANES)]
            obase = g * (G * D)
            for d in range(D):
                plsc.store_scatter(out_v, [iota * D + (obase + d)],
                                   accs[d] * inv_l)

        def outer(t, _):
            g0 = 2 * t
            dma(g0, buf_a, sem_a).wait()
            process(buf_a, g0)

            @pl.when(g0 + 2 < NG)
            def _():
                dma(g0 + 2, buf_a, sem_a).start()

            dma(g0 + 1, buf_b, sem_b).wait()
            process(buf_b, g0 + 1)

            @pl.when(g0 + 3 < NG)
            def _():
                dma(g0 + 3, buf_b, sem_b).start()

            return 0

        lax.fori_loop(0, NG // 2, outer, 0)
        pltpu.sync_copy(out_v, out_hbm.at[pl.ds(wid * (RPW * D), RPW * D)])

    return sc_kernel


def kernel(lookup_ids, table):
    B, L = lookup_ids.shape
    V, D = table.shape
    tab_rep = jnp.repeat(
        table.reshape(-1), LANES, total_repeat_length=V * D * LANES)
    out = _make_kernel(B, L, V, D)(lookup_ids.reshape(-1), tab_rep)
    return out.reshape(B, D)


# speedup vs baseline: 1.2081x; 1.2081x over previous
import functools

import jax
import jax.numpy as jnp
from jax import lax
from jax.experimental import pallas as pl
from jax.experimental.pallas import tpu as pltpu
from jax.experimental.pallas import tpu_sc as plsc

NC = 2
NS = 16
LANES = 16


@functools.lru_cache(maxsize=None)
def _make_kernel(B, L, V, D):
    NW = NC * NS
    RPW = B // NW
    G = LANES
    NG = RPW // G
    CHUNKS = L // LANES
    TAIL = L - CHUNKS * LANES
    assert B % (NW * G) == 0 and NG % 2 == 0 and L >= LANES
    CSTRIDE = 24
    assert CSTRIDE >= V and CSTRIDE % 8 == 0

    mesh = plsc.VectorSubcoreMesh(core_axis_name="c", subcore_axis_name="s")

    @functools.partial(
        pl.kernel,
        out_type=jax.ShapeDtypeStruct((B * D,), jnp.float32),
        mesh=mesh,
        compiler_params=pltpu.CompilerParams(needs_layout_passes=False),
        scratch_types=[
            pltpu.VMEM((G * L,), jnp.int32),
            pltpu.VMEM((G * L,), jnp.int32),
            pltpu.VMEM((V * D * LANES,), jnp.float32),
            pltpu.VMEM((G * CSTRIDE,), jnp.float32),
            pltpu.VMEM((RPW * D,), jnp.float32),
            pltpu.SemaphoreType.DMA,
            pltpu.SemaphoreType.DMA,
        ],
    )
    def sc_kernel(ids_hbm, tab_hbm, out_hbm,
                  buf_a, buf_b, tab_v, counts, out_v, sem_a, sem_b):
        wid = lax.axis_index("s") * NC + lax.axis_index("c")
        base = wid * (RPW * L)

        pltpu.sync_copy(tab_hbm, tab_v)

        iota = jnp.arange(LANES, dtype=jnp.int32)
        ones = jnp.full((LANES,), 1.0, jnp.float32)
        zeros = jnp.zeros((LANES,), jnp.float32)
        tail_mask = iota >= (LANES - TAIL)
        iota_c = iota * CSTRIDE
        iota_d = iota * D

        def dma(gi, buf, sem):
            return pltpu.make_async_copy(
                ids_hbm.at[pl.ds(base + gi * (G * L), G * L)], buf, sem)

        dma(0, buf_a, sem_a).start()
        dma(1, buf_b, sem_b).start()

        def process(buf, g):
            for k in range(G * CSTRIDE // LANES):
                counts[pl.ds(k * LANES, LANES)] = zeros

            @plsc.parallel_loop(0, G, unroll=2)
            def _(r):
                roff = r * L
                cref = counts.at[pl.ds(r * CSTRIDE, CSTRIDE)]
                for ci in range(CHUNKS):
                    chunk = buf[pl.ds(roff + ci * LANES, LANES)]
                    plsc.addupdate_scatter(cref, [chunk], ones)
                if TAIL:
                    tchunk = buf[pl.ds(roff + L - LANES, LANES)]
                    plsc.addupdate_scatter(cref, [tchunk], ones,
                                           mask=tail_mask)

            accs = [zeros] * D
            for v in range(V):
                row = plsc.load_gather(counts, [iota_c + v])
                for d in range(D):
                    accs[d] = accs[d] + row * tab_v[
                        pl.ds((v * D + d) * LANES, LANES)]
            obase = g * (G * D)
            for d in range(D):
                plsc.store_scatter(out_v.at[pl.ds(obase, G * D)],
                                   [iota_d + d], accs[d])

        def outer(t, _):
            g0 = 2 * t
            dma(g0, buf_a, sem_a).wait()
            process(buf_a, g0)

            @pl.when(g0 + 2 < NG)
            def _():
                dma(g0 + 2, buf_a, sem_a).start()

            dma(g0 + 1, buf_b, sem_b).wait()
            process(buf_b, g0 + 1)

            @pl.when(g0 + 3 < NG)
            def _():
                dma(g0 + 3, buf_b, sem_b).start()

            return 0

        lax.fori_loop(0, NG // 2, outer, 0)
        pltpu.sync_copy(out_v, out_hbm.at[pl.ds(wid * (RPW * D), RPW * D)])

    return sc_kernel


def kernel(lookup_ids, table):
    B, L = lookup_ids.shape
    V, D = table.shape
    tab_rep = jnp.repeat(
        (table / L).reshape(-1), LANES, total_repeat_length=V * D * LANES)
    out = _make_kernel(B, L, V, D)(lookup_ids.reshape(-1), tab_rep)
    return out.reshape(B, D)
